# Initial kernel scaffold; baseline (speedup 1.0000x reference)
import jax
import jax.numpy as jnp
from jax.experimental import pallas as pl


def kernel(y, contributors, weights, angles, kernel, center_kernel, bias):
    raise NotImplementedError("write your pallas kernel here")


# speedup vs baseline: 7.2729x; 7.2729x over previous
import functools

import jax
import jax.numpy as jnp
from jax import lax
from jax.experimental import pallas as pl
from jax.experimental.pallas import tpu as pltpu
from jax.experimental.pallas import tpu_sc as plsc

NV = 10000
NRINGS = 8
NDIRS = 8
NCH = 16
NFILTERS = 16

N = NV * NRINGS * NDIRS
NSUB = 16
ROWS_PER_W = N // NSUB
CHUNK = 1600
NCHUNKS = ROWS_PER_W // CHUNK
GROUPS = CHUNK // 16


def _sc_gather_body(y_hbm, ctr_hbm, w_hbm, z_hbm, y_v, ctr_v, w_v, out_v):
    h = lax.axis_index("c")
    rt = lax.axis_index("s")

    pltpu.sync_copy(y_hbm.at[h], y_v)

    iota = lax.iota(jnp.int32, 16)
    iota3 = iota * 3
    iota8 = iota * 8

    def chunk_body(j, carry):
        base_n = rt * ROWS_PER_W + j * CHUNK
        off3 = pl.multiple_of(base_n * 3, 8)
        pltpu.sync_copy(ctr_hbm.at[pl.ds(off3, CHUNK * 3)], ctr_v)
        pltpu.sync_copy(w_hbm.at[pl.ds(off3, CHUNK * 3)], w_v)

        def group_body(i, carry2):
            b3 = i * 48
            idx0 = iota3 + b3
            idx1 = idx0 + 1
            idx2 = idx0 + 2
            c0 = plsc.load_gather(ctr_v, [idx0]) * 8
            c1 = plsc.load_gather(ctr_v, [idx1]) * 8
            c2 = plsc.load_gather(ctr_v, [idx2]) * 8
            w0 = plsc.load_gather(w_v, [idx0])
            w1 = plsc.load_gather(w_v, [idx1])
            w2 = plsc.load_gather(w_v, [idx2])
            soff = i * 128
            for ch in range(8):
                g0 = plsc.load_gather(y_v, [c0 + ch])
                g1 = plsc.load_gather(y_v, [c1 + ch])
                g2 = plsc.load_gather(y_v, [c2 + ch])
                acc = w0 * g0 + w1 * g1 + w2 * g2
                plsc.store_scatter(out_v, [iota8 + (soff + ch)], acc)
            return carry2

        lax.fori_loop(0, GROUPS, group_body, 0)
        off8 = pl.multiple_of(base_n * 8, 8)
        pltpu.sync_copy(out_v, z_hbm.at[h, pl.ds(off8, CHUNK * 8)])
        return carry

    lax.fori_loop(0, NCHUNKS, chunk_body, 0)


@jax.jit
def _sc_gather(y_halves, ctr_flat, w_flat):
    mesh = plsc.VectorSubcoreMesh(core_axis_name="c", subcore_axis_name="s")
    f = pl.kernel(
        _sc_gather_body,
        out_type=jax.ShapeDtypeStruct((2, N * 8), jnp.float32),
        mesh=mesh,
        scratch_types=[
            pltpu.VMEM((NV * 8,), jnp.float32),
            pltpu.VMEM((CHUNK * 3,), jnp.int32),
            pltpu.VMEM((CHUNK * 3,), jnp.float32),
            pltpu.VMEM((CHUNK * 8,), jnp.float32),
        ],
        compiler_params=pltpu.CompilerParams(needs_layout_passes=False),
    )
    return f(y_halves, ctr_flat, w_flat)


R_BLK = 1000


def _tc_body(z0, z1, yb, k0, k1, ck, bb, ob):
    acc = jnp.dot(z0[...], k0[...], preferred_element_type=jnp.float32)
    acc = acc + jnp.dot(z1[...], k1[...], preferred_element_type=jnp.float32)
    yc = jnp.dot(yb[...], ck[...], preferred_element_type=jnp.float32)
    m = acc[:, 0:16]
    for dd in range(1, 8):
        m = jnp.maximum(m, acc[:, dd * 16:(dd + 1) * 16])
    ob[...] = jnp.maximum(m + yc + bb[...], 0.0)


@jax.jit
def _tc_conv(z0, z1, y2, K0, K1, ck, bias2):
    grid = (NV // R_BLK,)
    return pl.pallas_call(
        _tc_body,
        grid=grid,
        in_specs=[
            pl.BlockSpec((R_BLK, 512), lambda i: (i, 0)),
            pl.BlockSpec((R_BLK, 512), lambda i: (i, 0)),
            pl.BlockSpec((R_BLK, NCH), lambda i: (i, 0)),
            pl.BlockSpec((512, 128), lambda i: (0, 0)),
            pl.BlockSpec((512, 128), lambda i: (0, 0)),
            pl.BlockSpec((NCH, NFILTERS), lambda i: (0, 0)),
            pl.BlockSpec((1, NFILTERS), lambda i: (0, 0)),
        ],
        out_specs=pl.BlockSpec((R_BLK, NFILTERS), lambda i: (i, 0)),
        out_shape=jax.ShapeDtypeStruct((NV, NFILTERS), jnp.float32),
    )(z0, z1, y2, K0, K1, ck, bias2)


def kernel(y, contributors, weights, angles, kernel, center_kernel, bias):
    del angles
    y2 = y[0]
    y_halves = y2.reshape(NV, 2, 8).transpose(1, 0, 2).reshape(2, NV * 8)
    ctr_flat = contributors.reshape(-1)
    w_flat = weights.reshape(-1)

    Kb = jnp.stack([jnp.roll(kernel, dd, axis=1) for dd in range(NDIRS)],
                   axis=-2)
    Kb = Kb.reshape(NRINGS, NDIRS, NCH, NDIRS * NFILTERS)
    K0 = Kb[:, :, :8].reshape(512, 128)
    K1 = Kb[:, :, 8:].reshape(512, 128)

    z = _sc_gather(y_halves, ctr_flat, w_flat)
    z0 = z[0].reshape(NV, 512)
    z1 = z[1].reshape(NV, 512)
    out = _tc_conv(z0, z1, y2, K0, K1, center_kernel,
                   bias.reshape(1, NFILTERS))
    return out[None]


# speedup vs baseline: 61.7331x; 8.4882x over previous
import jax
import jax.numpy as jnp
from jax import lax
from jax.experimental import pallas as pl
from jax.experimental.pallas import tpu as pltpu
from jax.experimental.pallas import tpu_sc as plsc

NV = 10000
NRINGS = 8
NDIRS = 8
NCH = 16
NFILTERS = 16

VCHUNK = 128
NFULL = NV // VCHUNK
VTAIL = NV - NFULL * VCHUNK
ZCOLS = 260


def _sc_gather_body(yq_hbm, ctr_hbm, w_hbm, z_hbm, y_v, ctr_v, w_v, out_v):
    h = lax.axis_index("c")
    s = lax.axis_index("s")
    q = 2 * h + lax.rem(s, 2)
    g = lax.div(s, 2)

    pltpu.sync_copy(yq_hbm.at[q], y_v)

    iota = lax.iota(jnp.int32, 16)
    rowq = [jnp.broadcast_to(jnp.int32(ct), (16,)) for ct in range(4)]

    def compute_groups(v0, ngroups):
        def r_body(r, carry):
            def dg_body(dg, carry2):
                d = lax.div(dg, ngroups)
                gg = lax.rem(dg, ngroups)
                base = gg * 16
                rowidx = iota + base
                c0 = ctr_v[0, d, pl.ds(base, 16)]
                c1 = ctr_v[1, d, pl.ds(base, 16)]
                c2 = ctr_v[2, d, pl.ds(base, 16)]
                w0 = w_v[0, d, pl.ds(base, 16)]
                w1 = w_v[1, d, pl.ds(base, 16)]
                w2 = w_v[2, d, pl.ds(base, 16)]
                colbase = (r * 8 + d) * 4
                for ct in range(4):
                    g0 = plsc.load_gather(y_v, [rowq[ct], c0])
                    g1 = plsc.load_gather(y_v, [rowq[ct], c1])
                    g2 = plsc.load_gather(y_v, [rowq[ct], c2])
                    acc = w0 * g0 + w1 * g1 + w2 * g2
                    col = jnp.broadcast_to(colbase + ct, (16,))
                    plsc.store_scatter(out_v, [rowidx, col], acc)
                return carry2

            lax.fori_loop(0, 8 * ngroups, dg_body, 0)
            return carry

        def cg_body(cg, carry2):
            gg = lax.rem(cg, ngroups)
            rowidx = iota + gg * 16
            for ct in range(4):
                yv16 = y_v[ct, pl.ds(v0 + gg * 16, 16)]
                col = jnp.broadcast_to(jnp.int32(256 + ct), (16,))
                plsc.store_scatter(out_v, [rowidx, col], yv16)
            return carry2

        lax.fori_loop(0, ngroups, cg_body, 0)
        return r_body

    def chunk_body(t, carry):
        ci = g + 8 * t

        @pl.when(ci < NFULL)
        def _():
            v0 = pl.multiple_of(ci * VCHUNK, VCHUNK)
            r_body = compute_groups(v0, 8)

            def r_full(r, carry2):
                pltpu.sync_copy(
                    ctr_hbm.at[pl.ds(3 * r, 3), :, pl.ds(v0, VCHUNK)], ctr_v)
                pltpu.sync_copy(
                    w_hbm.at[pl.ds(3 * r, 3), :, pl.ds(v0, VCHUNK)], w_v)
                return r_body(r, carry2)

            lax.fori_loop(0, NRINGS, r_full, 0)
            pltpu.sync_copy(out_v, z_hbm.at[q, pl.ds(v0, VCHUNK), :])
        return carry

    lax.fori_loop(0, -(-NFULL // 8), chunk_body, 0)

    @pl.when(g == 7)
    def _():
        v0 = pl.multiple_of(NFULL * VCHUNK, VCHUNK)
        r_body = compute_groups(v0, 1)

        def r_tail(r, carry2):
            pltpu.sync_copy(
                ctr_hbm.at[pl.ds(3 * r, 3), :, pl.ds(v0, VTAIL)],
                ctr_v.at[:, :, pl.ds(0, VTAIL)])
            pltpu.sync_copy(
                w_hbm.at[pl.ds(3 * r, 3), :, pl.ds(v0, VTAIL)],
                w_v.at[:, :, pl.ds(0, VTAIL)])
            return r_body(r, carry2)

        lax.fori_loop(0, NRINGS, r_tail, 0)
        pltpu.sync_copy(out_v.at[pl.ds(0, VTAIL), :],
                        z_hbm.at[q, pl.ds(v0, VTAIL), :])


@jax.jit
def _sc_gather(y_q, ctr_t, w_t):
    mesh = plsc.VectorSubcoreMesh(core_axis_name="c", subcore_axis_name="s")
    f = pl.kernel(
        _sc_gather_body,
        out_type=jax.ShapeDtypeStruct((4, NV, ZCOLS), jnp.float32),
        mesh=mesh,
        scratch_types=[
            pltpu.VMEM((4, NV), jnp.float32),
            pltpu.VMEM((3, 8, VCHUNK), jnp.int32),
            pltpu.VMEM((3, 8, VCHUNK), jnp.float32),
            pltpu.VMEM((VCHUNK, ZCOLS), jnp.float32),
        ],
        compiler_params=pltpu.CompilerParams(needs_layout_passes=False),
    )
    return f(y_q, ctr_t, w_t)


R_BLK = 1000


def _tc_body(z_ref, kb_ref, bb_ref, ob_ref):
    zb = z_ref[...]
    kb = kb_ref[...]
    acc = jnp.dot(zb[0], kb[0], preferred_element_type=jnp.float32)
    for qq in range(1, 4):
        acc = acc + jnp.dot(zb[qq], kb[qq],
                            preferred_element_type=jnp.float32)
    m = acc[:, 0:16]
    for dd in range(1, 8):
        m = jnp.maximum(m, acc[:, dd * 16:(dd + 1) * 16])
    ob_ref[...] = jnp.maximum(m + bb_ref[...], 0.0)


@jax.jit
def _tc_conv(z, Kbig, bias2):
    grid = (NV // R_BLK,)
    return pl.pallas_call(
        _tc_body,
        grid=grid,
        in_specs=[
            pl.BlockSpec((4, R_BLK, ZCOLS), lambda i: (0, i, 0)),
            pl.BlockSpec((4, ZCOLS, 128), lambda i: (0, 0, 0)),
            pl.BlockSpec((1, NFILTERS), lambda i: (0, 0)),
        ],
        out_specs=pl.BlockSpec((R_BLK, NFILTERS), lambda i: (i, 0)),
        out_shape=jax.ShapeDtypeStruct((NV, NFILTERS), jnp.float32),
    )(z, Kbig, bias2)


def kernel(y, contributors, weights, angles, kernel, center_kernel, bias):
    del angles

    y_q = jnp.transpose(y, (0, 2, 1)).reshape(4, 4, NV)
    ctr_t = jnp.transpose(contributors, (0, 2, 4, 3, 1)).reshape(24, 8, NV)
    w_t = jnp.transpose(weights, (0, 2, 4, 3, 1)).reshape(24, 8, NV)

    Kb = jnp.stack([jnp.roll(kernel, dd, axis=1) for dd in range(NDIRS)],
                   axis=-2)
    Kb = Kb.reshape(NRINGS, NDIRS, 4, 4, NDIRS * NFILTERS)
    Kb = jnp.transpose(Kb, (2, 0, 1, 3, 4)).reshape(4, 256, 128)
    ckt = jnp.tile(center_kernel.reshape(4, 4, 1, NFILTERS),
                   (1, 1, NDIRS, 1)).reshape(4, 4, 128)
    Kbig = jnp.concatenate([Kb, ckt], axis=1)

    z = _sc_gather(y_q, ctr_t, w_t)
    out = _tc_conv(z, Kbig, bias.reshape(1, NFILTERS))
    return out[None]


# speedup vs baseline: 112.0053x; 1.8143x over previous
import jax
import jax.numpy as jnp
from jax import lax
from jax.experimental import pallas as pl
from jax.experimental.pallas import tpu as pltpu
from jax.experimental.pallas import tpu_sc as plsc

NV = 10000
NRINGS = 8
NDIRS = 8
NCH = 16
NFILTERS = 16

VCHUNK = 128
NFULL = NV // VCHUNK
VTAIL = NV - NFULL * VCHUNK
ZCOLS = 260


def _sc_gather_body(yq_hbm, ctr_hbm, w_hbm, z_hbm, y_v, ctr_v, w_v, out_v,
                    sc0, sc1, sw0, sw1):
    h = lax.axis_index("c")
    s = lax.axis_index("s")
    q = 2 * h + lax.rem(s, 2)
    g = lax.div(s, 2)

    pltpu.sync_copy(yq_hbm.at[q], y_v)

    iota = lax.iota(jnp.int32, 16)
    rowq = [jnp.broadcast_to(jnp.int32(ct), (16,)) for ct in range(4)]
    semc = [sc0, sc1]
    semw = [sw0, sw1]

    def do_chunk(v0, L, ngroups):
        def start(r, b):
            if L == VCHUNK:
                cdst, wdst = ctr_v.at[b], w_v.at[b]
            else:
                cdst = ctr_v.at[b, :, :, pl.ds(0, L)]
                wdst = w_v.at[b, :, :, pl.ds(0, L)]
            hc = pltpu.async_copy(
                ctr_hbm.at[pl.ds(3 * r, 3), :, pl.ds(v0, L)], cdst, semc[b])
            hw = pltpu.async_copy(
                w_hbm.at[pl.ds(3 * r, 3), :, pl.ds(v0, L)], wdst, semw[b])
            return hc, hw

        handles = [None, None]
        handles[0] = start(0, 0)
        for r in range(NRINGS):
            b = r & 1
            if r + 1 < NRINGS:
                handles[1 - b] = start(r + 1, 1 - b)
            handles[b][0].wait()
            handles[b][1].wait()

            @plsc.parallel_loop(0, 8 * ngroups, unroll=2)
            def _(dg):
                d = lax.div(dg, ngroups)
                gg = lax.rem(dg, ngroups)
                base = gg * 16
                rowidx = iota + base
                c0 = ctr_v[b, 0, d, pl.ds(base, 16)]
                c1 = ctr_v[b, 1, d, pl.ds(base, 16)]
                c2 = ctr_v[b, 2, d, pl.ds(base, 16)]
                w0 = w_v[b, 0, d, pl.ds(base, 16)]
                w1 = w_v[b, 1, d, pl.ds(base, 16)]
                w2 = w_v[b, 2, d, pl.ds(base, 16)]
                colbase = r * 32 + d * 4
                for ct in range(4):
                    g0 = plsc.load_gather(y_v, [rowq[ct], c0])
                    g1 = plsc.load_gather(y_v, [rowq[ct], c1])
                    g2 = plsc.load_gather(y_v, [rowq[ct], c2])
                    acc = w0 * g0 + w1 * g1 + w2 * g2
                    col = jnp.broadcast_to(colbase + ct, (16,))
                    plsc.store_scatter(out_v, [rowidx, col], acc)

        @plsc.parallel_loop(0, ngroups)
        def _(gg):
            rowidx = iota + gg * 16
            for ct in range(4):
                yv16 = y_v[ct, pl.ds(v0 + gg * 16, 16)]
                col = jnp.broadcast_to(jnp.int32(256 + ct), (16,))
                plsc.store_scatter(out_v, [rowidx, col], yv16)

        if L == VCHUNK:
            pltpu.sync_copy(out_v, z_hbm.at[q, pl.ds(v0, VCHUNK), :])
        else:
            pltpu.sync_copy(out_v.at[pl.ds(0, L), :],
                            z_hbm.at[q, pl.ds(v0, L), :])

    def chunk_body(t, carry):
        ci = g + 8 * t

        @pl.when(ci < NFULL)
        def _():
            do_chunk(pl.multiple_of(ci * VCHUNK, VCHUNK), VCHUNK, 8)
        return carry

    lax.fori_loop(0, -(-NFULL // 8), chunk_body, 0)

    @pl.when(g == 7)
    def _():
        do_chunk(pl.multiple_of(NFULL * VCHUNK, VCHUNK), VTAIL, 1)


@jax.jit
def _sc_gather(y_q, ctr_t, w_t):
    mesh = plsc.VectorSubcoreMesh(core_axis_name="c", subcore_axis_name="s")
    f = pl.kernel(
        _sc_gather_body,
        out_type=jax.ShapeDtypeStruct((4, NV, ZCOLS), jnp.float32),
        mesh=mesh,
        scratch_types=[
            pltpu.VMEM((4, NV), jnp.float32),
            pltpu.VMEM((2, 3, 8, VCHUNK), jnp.int32),
            pltpu.VMEM((2, 3, 8, VCHUNK), jnp.float32),
            pltpu.VMEM((VCHUNK, ZCOLS), jnp.float32),
            pltpu.SemaphoreType.DMA,
            pltpu.SemaphoreType.DMA,
            pltpu.SemaphoreType.DMA,
            pltpu.SemaphoreType.DMA,
        ],
        compiler_params=pltpu.CompilerParams(needs_layout_passes=False),
    )
    return f(y_q, ctr_t, w_t)


R_BLK = 1000


def _tc_body(z_ref, kb_ref, bb_ref, ob_ref):
    zb = z_ref[...]
    kb = kb_ref[...]
    acc = jnp.dot(zb[0], kb[0], preferred_element_type=jnp.float32)
    for qq in range(1, 4):
        acc = acc + jnp.dot(zb[qq], kb[qq],
                            preferred_element_type=jnp.float32)
    m = acc[:, 0:16]
    for dd in range(1, 8):
        m = jnp.maximum(m, acc[:, dd * 16:(dd + 1) * 16])
    ob_ref[...] = jnp.maximum(m + bb_ref[...], 0.0)


@jax.jit
def _tc_conv(z, Kbig, bias2):
    grid = (NV // R_BLK,)
    return pl.pallas_call(
        _tc_body,
        grid=grid,
        in_specs=[
            pl.BlockSpec((4, R_BLK, ZCOLS), lambda i: (0, i, 0)),
            pl.BlockSpec((4, ZCOLS, 128), lambda i: (0, 0, 0)),
            pl.BlockSpec((1, NFILTERS), lambda i: (0, 0)),
        ],
        out_specs=pl.BlockSpec((R_BLK, NFILTERS), lambda i: (i, 0)),
        out_shape=jax.ShapeDtypeStruct((NV, NFILTERS), jnp.float32),
    )(z, Kbig, bias2)


def kernel(y, contributors, weights, angles, kernel, center_kernel, bias):
    del angles

    y_q = jnp.transpose(y, (0, 2, 1)).reshape(4, 4, NV)
    ctr_t = jnp.transpose(contributors, (0, 2, 4, 3, 1)).reshape(24, 8, NV)
    w_t = jnp.transpose(weights, (0, 2, 4, 3, 1)).reshape(24, 8, NV)

    Kb = jnp.stack([jnp.roll(kernel, dd, axis=1) for dd in range(NDIRS)],
                   axis=-2)
    Kb = Kb.reshape(NRINGS, NDIRS, 4, 4, NDIRS * NFILTERS)
    Kb = jnp.transpose(Kb, (2, 0, 1, 3, 4)).reshape(4, 256, 128)
    ckt = jnp.tile(center_kernel.reshape(4, 4, 1, NFILTERS),
                   (1, 1, NDIRS, 1)).reshape(4, 4, 128)
    Kbig = jnp.concatenate([Kb, ckt], axis=1)

    z = _sc_gather(y_q, ctr_t, w_t)
    out = _tc_conv(z, Kbig, bias.reshape(1, NFILTERS))
    return out[None]
